# baseline (speedup 1.0000x reference)
import jax
import jax.numpy as jnp
from jax import lax
from jax.experimental import pallas as pl
from jax.experimental.pallas import tpu as pltpu
from jax.experimental.pallas import tpu_sc as plsc

N = 10000
E = 320000
D = 128
H = 128
B = 256

NC = 2
NS = 16
L = 16
NW = NC * NS
EPW = E // NW
G = 80
NCHUNK = EPW // G
AW = H + L
RPS = 624
RTAIL = N - NS * RPS
RC = 16
NRC = N // RC
EPI_K = (NRC + NW - 1) // NW
SPW = B // NW
RSQRT_H = 0.08838834764831845
CPB = 25
NBLK = NCHUNK // CPB

_MESH = plsc.VectorSubcoreMesh(
    core_axis_name="c", subcore_axis_name="s", num_cores=NC, num_subcores=NS)
_SC_PARAMS = pltpu.CompilerParams(
    needs_layout_passes=False, use_tc_tiling_on_sc=False)


def _wid():
    return lax.axis_index("s") * NC + lax.axis_index("c")


def _proj_body(h_ref, w_ref, b_ref, q_ref, k_ref, v_ref, s_ref):
    acc = jnp.dot(h_ref[...], w_ref[...], preferred_element_type=jnp.float32)
    acc = acc + b_ref[...]
    q_ref[...] = acc[:, :H].astype(jnp.bfloat16)
    k_ref[...] = acc[:, H:2 * H].astype(jnp.bfloat16)
    v_ref[...] = acc[:, 2 * H:3 * H]
    s_ref[...] = acc[:, 3 * H:]


def _proj(h, wT, b2d):
    bn = 1000
    grid = (N // bn,)
    outb = jax.ShapeDtypeStruct((N, H), jnp.bfloat16)
    out = jax.ShapeDtypeStruct((N, H), jnp.float32)
    return pl.pallas_call(
        _proj_body,
        grid=grid,
        in_specs=[
            pl.BlockSpec((bn, D), lambda i: (i, 0)),
            pl.BlockSpec((D, 4 * H), lambda i: (0, 0)),
            pl.BlockSpec((1, 4 * H), lambda i: (0, 0)),
        ],
        out_specs=[pl.BlockSpec((bn, H), lambda i: (i, 0))] * 4,
        out_shape=[outb, outb, out, out],
    )(h, wT, b2d)


def _mlp_body(h_ref, w1_ref, b1_ref, w2_ref, b2_ref, w3_ref, b3_ref, out_ref):
    h = h_ref[...]
    h = jax.nn.relu(jnp.dot(h, w1_ref[...].T, preferred_element_type=jnp.float32) + b1_ref[...])
    h = jax.nn.relu(jnp.dot(h, w2_ref[...].T, preferred_element_type=jnp.float32) + b2_ref[...])
    z = jnp.sum(h * w3_ref[...], axis=-1, keepdims=True) + b3_ref[0]
    out_ref[...] = jax.nn.sigmoid(z)


def _mlp(h, W1, b1, W2, b2, W3, b3):
    return pl.pallas_call(
        _mlp_body,
        out_shape=jax.ShapeDtypeStruct((B, 1), jnp.float32),
    )(h, W1, b1, W2, b2, W3, b3)


def _score_body(q_hbm, k_hbm, src3_hbm, dst3_hbm, scores_hbm, pmax_hbm,
                si_v, di_v, qrA, krA, qrB, krB, qrC, krC, sc_v, mx_v,
                sqA, skA, sqB, skB, sqC, skC):
    wid = _wid()
    pltpu.sync_copy(src3_hbm.at[wid], si_v)
    pltpu.sync_copy(dst3_hbm.at[wid], di_v)

    def start(ci, qr, kr, sq, sk):
        pltpu.async_copy(q_hbm.at[di_v.at[ci]], qr, sq)
        pltpu.async_copy(k_hbm.at[si_v.at[ci]], kr, sk)

    def wait(qr, kr, sq, sk):
        pltpu.make_async_copy(q_hbm.at[di_v.at[0]], qr, sq).wait()
        pltpu.make_async_copy(k_hbm.at[si_v.at[0]], kr, sk).wait()

    def compute(ci, qr, kr, smax):
        sbase = ci * G

        def eloop(e2, smax):
            for u in range(2):
                e = e2 * 2 + u
                acc = jnp.zeros((L,), jnp.float32)
                for c in range(4):
                    q0, q1 = plsc.unpack(qr[e, pl.ds(c * 2 * L, 2 * L)],
                                         format=plsc.PackFormat.INTERLEAVED)
                    k0, k1 = plsc.unpack(kr[e, pl.ds(c * 2 * L, 2 * L)],
                                         format=plsc.PackFormat.INTERLEAVED)
                    acc = acc + q0 * k0 + q1 * k1
                s = jnp.sum(acc) * RSQRT_H
                plsc.store_scatter(sc_v, [jnp.full((L,), sbase + e, jnp.int32)],
                                   jnp.full((L,), s, jnp.float32))
                smax = jnp.maximum(smax, s)
            return smax

        return lax.fori_loop(0, G // 2, eloop, smax)

    start(0, qrA, krA, sqA, skA)
    start(1, qrB, krB, sqB, skB)

    def tstep(i, smax):
        c0 = 3 * i
        start(c0 + 2, qrC, krC, sqC, skC)
        wait(qrA, krA, sqA, skA)
        smax = compute(c0, qrA, krA, smax)
        start(c0 + 3, qrA, krA, sqA, skA)
        wait(qrB, krB, sqB, skB)
        smax = compute(c0 + 1, qrB, krB, smax)
        start(c0 + 4, qrB, krB, sqB, skB)
        wait(qrC, krC, sqC, skC)
        smax = compute(c0 + 2, qrC, krC, smax)
        return smax

    smax = lax.fori_loop(0, (NCHUNK - 2) // 3, tstep, jnp.float32(-3.0e38))
    wait(qrA, krA, sqA, skA)
    smax = compute(NCHUNK - 2, qrA, krA, smax)
    wait(qrB, krB, sqB, skB)
    smax = compute(NCHUNK - 1, qrB, krB, smax)
    mx_v[...] = jnp.full((L,), smax, jnp.float32)
    pltpu.sync_copy(sc_v, scores_hbm.at[wid])
    pltpu.sync_copy(mx_v, pmax_hbm.at[wid])


def _score(q, k, src3, dst3):
    f = pl.kernel(
        _score_body,
        out_type=[
            jax.ShapeDtypeStruct((NW, EPW), jnp.float32),
            jax.ShapeDtypeStruct((NW, L), jnp.float32),
        ],
        mesh=_MESH,
        compiler_params=_SC_PARAMS,
        scratch_types=[
            pltpu.VMEM((NCHUNK, G), jnp.int32),
            pltpu.VMEM((NCHUNK, G), jnp.int32),
            pltpu.VMEM((G, H), jnp.bfloat16),
            pltpu.VMEM((G, H), jnp.bfloat16),
            pltpu.VMEM((G, H), jnp.bfloat16),
            pltpu.VMEM((G, H), jnp.bfloat16),
            pltpu.VMEM((G, H), jnp.bfloat16),
            pltpu.VMEM((G, H), jnp.bfloat16),
            pltpu.VMEM((EPW,), jnp.float32),
            pltpu.VMEM((L,), jnp.float32),
            pltpu.SemaphoreType.DMA,
            pltpu.SemaphoreType.DMA,
            pltpu.SemaphoreType.DMA,
            pltpu.SemaphoreType.DMA,
            pltpu.SemaphoreType.DMA,
            pltpu.SemaphoreType.DMA,
        ],
    )
    return f(q, k, src3, dst3)


GA = 32
GB = 48


def _agg_body(v_hbm, src3_hbm, dstA_hbm, dstB_hbm, scores_hbm, pmax_hbm,
              zeros_hbm, accp_hbm,
              si_v, diA_v, diB_v, sc_v, es_v, vrA, vrB, vwA, vwB, pm_v, acc_sh,
              semA, semB, ssA, ssB):
    cid = lax.axis_index("c")
    sid = lax.axis_index("s")
    wid = sid * NC + cid
    pltpu.sync_copy(pmax_hbm, pm_v)
    m = pm_v[0, :]
    for r in range(1, NW):
        m = jnp.maximum(m, pm_v[r, :])
    gmax = jnp.max(m)
    rows0 = pl.multiple_of(sid * RPS, 8)
    pltpu.sync_copy(zeros_hbm.at[pl.ds(rows0, RPS)], acc_sh.at[pl.ds(rows0, RPS)])

    @pl.when(sid == NS - 1)
    def _zero_tail():
        t0 = pl.multiple_of(NS * RPS, 8)
        pltpu.sync_copy(zeros_hbm.at[pl.ds(t0, RTAIL)], acc_sh.at[pl.ds(t0, RTAIL)])

    plsc.subcore_barrier()

    def start(ci, vr, sem):
        pltpu.async_copy(v_hbm.at[si_v.at[ci]], vr, sem)

    def wait(vr, sem):
        pltpu.make_async_copy(v_hbm.at[si_v.at[0]], vr, sem).wait()

    def wait_sc(vw, ss, nr):
        pltpu.make_async_copy(zeros_hbm.at[pl.ds(0, nr)], vw, ss).wait()

    def half(gci, ci, goff, ng, vr, vw, ss, di, nr):
        @pl.when(gci > 0)
        def _():
            wait_sc(vw, ss, nr)

        def eloop(gi, c):
            g = goff + gi
            ev = es_v[pl.ds(g * L, L)]
            for u in range(L):
                e = g * L + u
                esc = ev[u]
                w = gi * L + u
                for cc in range(8):
                    vw[w, pl.ds(cc * L, L)] = vr[e, pl.ds(cc * L, L)] * esc
                vw[w, pl.ds(H, L)] = jnp.full((L,), esc, jnp.float32)
            return c

        lax.fori_loop(0, ng, eloop, 0)
        pltpu.async_copy(vw, acc_sh.at[di.at[ci]], ss, add=True)

    def compute(bo, ci, vr):
        gci = bo * CPB + ci
        for g in range(G // L):
            es_v[pl.ds(g * L, L)] = jnp.exp(
                sc_v[pl.ds(ci * G + g * L, L)] - gmax)
        half(gci, ci, 0, GA // L, vr, vwA, ssA, diA_v, GA)
        half(gci, ci, GA // L, GB // L, vr, vwB, ssB, diB_v, GB)

    def block(bo, carry):
        b0 = bo * CPB
        pltpu.sync_copy(src3_hbm.at[wid, pl.ds(b0, CPB)], si_v)
        pltpu.sync_copy(dstA_hbm.at[wid, pl.ds(b0, CPB)], diA_v)
        pltpu.sync_copy(dstB_hbm.at[wid, pl.ds(b0, CPB)], diB_v)
        sc0 = pl.multiple_of(b0 * G, 8)
        pltpu.sync_copy(scores_hbm.at[wid, pl.ds(sc0, CPB * G)], sc_v)
        start(0, vrA, semA)

        def dstep(i, c):
            c0 = 2 * i
            start(c0 + 1, vrB, semB)
            wait(vrA, semA)
            compute(bo, c0, vrA)
            start(c0 + 2, vrA, semA)
            wait(vrB, semB)
            compute(bo, c0 + 1, vrB)
            return c

        lax.fori_loop(0, (CPB - 1) // 2, dstep, 0)
        wait(vrA, semA)
        compute(bo, CPB - 1, vrA)
        return carry

    lax.fori_loop(0, NBLK, block, 0)
    wait_sc(vwA, ssA, GA)
    wait_sc(vwB, ssB, GB)
    plsc.subcore_barrier()
    pltpu.sync_copy(acc_sh.at[pl.ds(rows0, RPS)],
                    accp_hbm.at[cid, pl.ds(rows0, RPS)])

    @pl.when(sid == NS - 1)
    def _wb_tail():
        t0 = pl.multiple_of(NS * RPS, 8)
        pltpu.sync_copy(acc_sh.at[pl.ds(t0, RTAIL)],
                        accp_hbm.at[cid, pl.ds(t0, RTAIL)])


def _agg(v, src3, dstA, dstB, scores, pmax, zeros):
    f = pl.kernel(
        _agg_body,
        out_type=jax.ShapeDtypeStruct((NC, N, AW), jnp.float32),
        mesh=_MESH,
        compiler_params=_SC_PARAMS,
        scratch_types=[
            pltpu.VMEM((CPB, G), jnp.int32),
            pltpu.VMEM((CPB, GA), jnp.int32),
            pltpu.VMEM((CPB, GB), jnp.int32),
            pltpu.VMEM((CPB * G,), jnp.float32),
            pltpu.VMEM((G,), jnp.float32),
            pltpu.VMEM((G, H), jnp.float32),
            pltpu.VMEM((G, H), jnp.float32),
            pltpu.VMEM((GA, AW), jnp.float32),
            pltpu.VMEM((GB, AW), jnp.float32),
            pltpu.VMEM((NW, L), jnp.float32),
            pltpu.VMEM_SHARED((N, AW), jnp.float32),
            pltpu.SemaphoreType.DMA,
            pltpu.SemaphoreType.DMA,
            pltpu.SemaphoreType.DMA,
            pltpu.SemaphoreType.DMA,
        ],
    )
    return f(v, src3, dstA, dstB, scores, pmax, zeros)


def _epi_body(accp_hbm, s_hbm, h_hbm, a0_v, a1_v, sv_v, hv_v, s0, s1, s2):
    wid = _wid()

    def step(k, carry):
        t = wid + k * NW

        @pl.when(t < NRC)
        def _():
            r0 = pl.multiple_of(t * RC, 8)
            cp0 = pltpu.async_copy(accp_hbm.at[0, pl.ds(r0, RC)], a0_v, s0)
            cp1 = pltpu.async_copy(accp_hbm.at[1, pl.ds(r0, RC)], a1_v, s1)
            cp2 = pltpu.async_copy(s_hbm.at[pl.ds(r0, RC)], sv_v, s2)
            cp0.wait()
            cp1.wait()
            cp2.wait()
            def rloop(r, c):
                d0 = a0_v[r, pl.ds(H, L)]
                d1 = a1_v[r, pl.ds(H, L)]
                rden = 1.0 / (d0 + d1 + 1e-16)
                for cc in range(8):
                    num = (a0_v[r, pl.ds(cc * L, L)]
                           + a1_v[r, pl.ds(cc * L, L)])
                    hv_v[r, pl.ds(cc * L, L)] = jnp.maximum(
                        sv_v[r, pl.ds(cc * L, L)] + num * rden, 0.0)
                return c

            lax.fori_loop(0, RC, rloop, 0)
            pltpu.sync_copy(hv_v, h_hbm.at[pl.ds(r0, RC)])

        return carry

    lax.fori_loop(0, EPI_K, step, 0)


def _epi(accp, s):
    f = pl.kernel(
        _epi_body,
        out_type=jax.ShapeDtypeStruct((N, H), jnp.float32),
        mesh=_MESH,
        compiler_params=_SC_PARAMS,
        scratch_types=[
            pltpu.VMEM((RC, AW), jnp.float32),
            pltpu.VMEM((RC, AW), jnp.float32),
            pltpu.VMEM((RC, H), jnp.float32),
            pltpu.VMEM((RC, H), jnp.float32),
            pltpu.SemaphoreType.DMA,
            pltpu.SemaphoreType.DMA,
            pltpu.SemaphoreType.DMA,
        ],
    )
    return f(accp, s)


def _sel_body(h_hbm, idx_hbm, out_hbm, idx_v, rows_v, sem):
    wid = _wid()
    base = pl.multiple_of(wid * SPW, 8)
    pltpu.sync_copy(idx_hbm.at[pl.ds(base, SPW)], idx_v)
    pltpu.async_copy(h_hbm.at[idx_v], rows_v, sem).wait()
    pltpu.sync_copy(rows_v, out_hbm.at[pl.ds(base, SPW)])


def _sel(h, idx):
    f = pl.kernel(
        _sel_body,
        out_type=jax.ShapeDtypeStruct((B, H), jnp.float32),
        mesh=_MESH,
        compiler_params=_SC_PARAMS,
        scratch_types=[
            pltpu.VMEM((SPW,), jnp.int32),
            pltpu.VMEM((SPW, H), jnp.float32),
            pltpu.SemaphoreType.DMA,
        ],
    )
    return f(h, idx)


def _layer(h, src3, dst3, dstA, dstB, wT, b2d, zeros):
    q, k, v, s = _proj(h, wT, b2d)
    scores, pmax = _score(q, k, src3, dst3)
    accp = _agg(v, src3, dstA, dstB, scores, pmax, zeros)
    return _epi(accp, s)


def kernel(x, edge_index, idx,
           Wq0, bq0, Wk0, bk0, Wv0, bv0, Ws0, bs0,
           Wq1, bq1, Wk1, bk1, Wv1, bv1, Ws1, bs1,
           W1, b1, W2, b2, W3, b3):
    src3 = edge_index[0].reshape(NW, NCHUNK, G)
    dst3 = edge_index[1].reshape(NW, NCHUNK, G)
    dstA = dst3[:, :, :GA]
    dstB = dst3[:, :, GA:]
    zeros = jnp.zeros((N, AW), jnp.float32)
    w0T = jnp.concatenate([Wq0, Wk0, Wv0, Ws0], axis=0).T
    b0 = jnp.concatenate([bq0, bk0, bv0, bs0]).reshape(1, 4 * H)
    w1T = jnp.concatenate([Wq1, Wk1, Wv1, Ws1], axis=0).T
    b1c = jnp.concatenate([bq1, bk1, bv1, bs1]).reshape(1, 4 * H)
    h = _layer(x, src3, dst3, dstA, dstB, w0T, b0, zeros)
    h = _layer(h, src3, dst3, dstA, dstB, w1T, b1c, zeros)
    hsel = _sel(h, idx)
    return _mlp(hsel, W1, b1, W2, b2, W3, b3)


# speedup vs baseline: 1.2733x; 1.2733x over previous
import jax
import jax.numpy as jnp
from jax import lax
from jax.experimental import pallas as pl
from jax.experimental.pallas import tpu as pltpu
from jax.experimental.pallas import tpu_sc as plsc

N = 10000
E = 320000
D = 128
H = 128
B = 256

NC = 2
NS = 16
L = 16
NW = NC * NS
EPW = E // NW
G = 80
NCHUNK = EPW // G
AW = H + L
RPS = 624
RTAIL = N - NS * RPS
RC = 16
NRC = N // RC
EPI_K = (NRC + NW - 1) // NW
SPW = B // NW
RSQRT_H = 0.08838834764831845
CPB = 25
NBLK = NCHUNK // CPB

_MESH = plsc.VectorSubcoreMesh(
    core_axis_name="c", subcore_axis_name="s", num_cores=NC, num_subcores=NS)
_SC_PARAMS = pltpu.CompilerParams(
    needs_layout_passes=False, use_tc_tiling_on_sc=False)


def _wid():
    return lax.axis_index("s") * NC + lax.axis_index("c")


def _proj_body(h_ref, w_ref, b_ref, q_ref, k_ref, v_ref, s_ref):
    acc = jnp.dot(h_ref[...], w_ref[...], preferred_element_type=jnp.float32)
    acc = acc + b_ref[...]
    q_ref[...] = acc[:, :H]
    k_ref[...] = acc[:, H:2 * H]
    v_ref[...] = acc[:, 2 * H:3 * H]
    s_ref[...] = acc[:, 3 * H:]


def _proj(h, wT, b2d):
    bn = 1000
    grid = (N // bn,)
    outb = jax.ShapeDtypeStruct((N, H), jnp.float32)
    out = jax.ShapeDtypeStruct((N, H), jnp.float32)
    return pl.pallas_call(
        _proj_body,
        grid=grid,
        in_specs=[
            pl.BlockSpec((bn, D), lambda i: (i, 0)),
            pl.BlockSpec((D, 4 * H), lambda i: (0, 0)),
            pl.BlockSpec((1, 4 * H), lambda i: (0, 0)),
        ],
        out_specs=[pl.BlockSpec((bn, H), lambda i: (i, 0))] * 4,
        out_shape=[outb, outb, out, out],
    )(h, wT, b2d)


def _mlp_body(h_ref, w1_ref, b1_ref, w2_ref, b2_ref, w3_ref, b3_ref, out_ref):
    h = h_ref[...]
    h = jax.nn.relu(jnp.dot(h, w1_ref[...].T, preferred_element_type=jnp.float32) + b1_ref[...])
    h = jax.nn.relu(jnp.dot(h, w2_ref[...].T, preferred_element_type=jnp.float32) + b2_ref[...])
    z = jnp.sum(h * w3_ref[...], axis=-1, keepdims=True) + b3_ref[0]
    out_ref[...] = jax.nn.sigmoid(z)


def _mlp(h, W1, b1, W2, b2, W3, b3):
    return pl.pallas_call(
        _mlp_body,
        out_shape=jax.ShapeDtypeStruct((B, 1), jnp.float32),
    )(h, W1, b1, W2, b2, W3, b3)


def _score_body(q_hbm, k_hbm, src3_hbm, dst3_hbm, scores_hbm, pmax_hbm,
                si_v, di_v, qrA, krA, qrB, krB, qrC, krC, sc_v, mx_v,
                sqA, skA, sqB, skB, sqC, skC):
    wid = _wid()
    pltpu.sync_copy(src3_hbm.at[wid], si_v)
    pltpu.sync_copy(dst3_hbm.at[wid], di_v)

    def start(ci, qr, kr, sq, sk):
        pltpu.async_copy(q_hbm.at[di_v.at[ci]], qr, sq)
        pltpu.async_copy(k_hbm.at[si_v.at[ci]], kr, sk)

    def wait(qr, kr, sq, sk):
        pltpu.make_async_copy(q_hbm.at[di_v.at[0]], qr, sq).wait()
        pltpu.make_async_copy(k_hbm.at[si_v.at[0]], kr, sk).wait()

    def compute(ci, qr, kr, smax):
        sbase = ci * G

        def eloop(e4, smax):
            svals = []
            for u in range(4):
                e = e4 * 4 + u
                acc = qr[e, pl.ds(0, L)] * kr[e, pl.ds(0, L)]
                for c in range(1, 8):
                    acc = acc + qr[e, pl.ds(c * L, L)] * kr[e, pl.ds(c * L, L)]
                svals.append(jnp.sum(acc) * RSQRT_H)
            for u in range(4):
                e = e4 * 4 + u
                s = svals[u]
                plsc.store_scatter(sc_v, [jnp.full((L,), sbase + e, jnp.int32)],
                                   jnp.full((L,), s, jnp.float32))
                smax = jnp.maximum(smax, s)
            return smax

        return lax.fori_loop(0, G // 4, eloop, smax)

    start(0, qrA, krA, sqA, skA)
    start(1, qrB, krB, sqB, skB)

    def tstep(i, smax):
        c0 = 3 * i
        start(c0 + 2, qrC, krC, sqC, skC)
        wait(qrA, krA, sqA, skA)
        smax = compute(c0, qrA, krA, smax)
        start(c0 + 3, qrA, krA, sqA, skA)
        wait(qrB, krB, sqB, skB)
        smax = compute(c0 + 1, qrB, krB, smax)
        start(c0 + 4, qrB, krB, sqB, skB)
        wait(qrC, krC, sqC, skC)
        smax = compute(c0 + 2, qrC, krC, smax)
        return smax

    smax = lax.fori_loop(0, (NCHUNK - 2) // 3, tstep, jnp.float32(-3.0e38))
    wait(qrA, krA, sqA, skA)
    smax = compute(NCHUNK - 2, qrA, krA, smax)
    wait(qrB, krB, sqB, skB)
    smax = compute(NCHUNK - 1, qrB, krB, smax)
    mx_v[...] = jnp.full((L,), smax, jnp.float32)
    pltpu.sync_copy(sc_v, scores_hbm.at[wid])
    pltpu.sync_copy(mx_v, pmax_hbm.at[wid])


def _score(q, k, src3, dst3):
    f = pl.kernel(
        _score_body,
        out_type=[
            jax.ShapeDtypeStruct((NW, EPW), jnp.float32),
            jax.ShapeDtypeStruct((NW, L), jnp.float32),
        ],
        mesh=_MESH,
        compiler_params=_SC_PARAMS,
        scratch_types=[
            pltpu.VMEM((NCHUNK, G), jnp.int32),
            pltpu.VMEM((NCHUNK, G), jnp.int32),
            pltpu.VMEM((G, H), jnp.float32),
            pltpu.VMEM((G, H), jnp.float32),
            pltpu.VMEM((G, H), jnp.float32),
            pltpu.VMEM((G, H), jnp.float32),
            pltpu.VMEM((G, H), jnp.float32),
            pltpu.VMEM((G, H), jnp.float32),
            pltpu.VMEM((EPW,), jnp.float32),
            pltpu.VMEM((L,), jnp.float32),
            pltpu.SemaphoreType.DMA,
            pltpu.SemaphoreType.DMA,
            pltpu.SemaphoreType.DMA,
            pltpu.SemaphoreType.DMA,
            pltpu.SemaphoreType.DMA,
            pltpu.SemaphoreType.DMA,
        ],
    )
    return f(q, k, src3, dst3)


GA = 32
GB = 48


def _agg_body(v_hbm, src3_hbm, dstA_hbm, dstB_hbm, scores_hbm, pmax_hbm,
              zeros_hbm, accp_hbm,
              si_v, diA_v, diB_v, sc_v, es_v, vrA, vrB, vwA, vwB, pm_v, acc_sh,
              semA, semB, ssA, ssB):
    cid = lax.axis_index("c")
    sid = lax.axis_index("s")
    wid = sid * NC + cid
    pltpu.sync_copy(pmax_hbm, pm_v)
    m = pm_v[0, :]
    for r in range(1, NW):
        m = jnp.maximum(m, pm_v[r, :])
    gmax = jnp.max(m)
    rows0 = pl.multiple_of(sid * RPS, 8)
    pltpu.sync_copy(zeros_hbm.at[pl.ds(rows0, RPS)], acc_sh.at[pl.ds(rows0, RPS)])

    @pl.when(sid == NS - 1)
    def _zero_tail():
        t0 = pl.multiple_of(NS * RPS, 8)
        pltpu.sync_copy(zeros_hbm.at[pl.ds(t0, RTAIL)], acc_sh.at[pl.ds(t0, RTAIL)])

    plsc.subcore_barrier()

    def start(ci, vr, sem):
        pltpu.async_copy(v_hbm.at[si_v.at[ci]], vr, sem)

    def wait(vr, sem):
        pltpu.make_async_copy(v_hbm.at[si_v.at[0]], vr, sem).wait()

    def wait_sc(vw, ss, nr):
        pltpu.make_async_copy(zeros_hbm.at[pl.ds(0, nr)], vw, ss).wait()

    def half(gci, ci, goff, ng, vr, vw, ss, di, nr):
        @pl.when(gci > 0)
        def _():
            wait_sc(vw, ss, nr)

        def eloop(gi, c):
            g = goff + gi
            ev = es_v[pl.ds(g * L, L)]
            for u in range(L):
                e = g * L + u
                esc = ev[u]
                w = gi * L + u
                for cc in range(8):
                    vw[w, pl.ds(cc * L, L)] = vr[e, pl.ds(cc * L, L)] * esc
                vw[w, pl.ds(H, L)] = jnp.full((L,), esc, jnp.float32)
            return c

        lax.fori_loop(0, ng, eloop, 0)
        pltpu.async_copy(vw, acc_sh.at[di.at[ci]], ss, add=True)

    def compute(bo, ci, vr):
        gci = bo * CPB + ci
        for g in range(G // L):
            es_v[pl.ds(g * L, L)] = jnp.exp(
                sc_v[pl.ds(ci * G + g * L, L)] - gmax)
        half(gci, ci, 0, GA // L, vr, vwA, ssA, diA_v, GA)
        half(gci, ci, GA // L, GB // L, vr, vwB, ssB, diB_v, GB)

    def block(bo, carry):
        b0 = bo * CPB
        pltpu.sync_copy(src3_hbm.at[wid, pl.ds(b0, CPB)], si_v)
        pltpu.sync_copy(dstA_hbm.at[wid, pl.ds(b0, CPB)], diA_v)
        pltpu.sync_copy(dstB_hbm.at[wid, pl.ds(b0, CPB)], diB_v)
        sc0 = pl.multiple_of(b0 * G, 8)
        pltpu.sync_copy(scores_hbm.at[wid, pl.ds(sc0, CPB * G)], sc_v)
        start(0, vrA, semA)

        def dstep(i, c):
            c0 = 2 * i
            start(c0 + 1, vrB, semB)
            wait(vrA, semA)
            compute(bo, c0, vrA)
            start(c0 + 2, vrA, semA)
            wait(vrB, semB)
            compute(bo, c0 + 1, vrB)
            return c

        lax.fori_loop(0, (CPB - 1) // 2, dstep, 0)
        wait(vrA, semA)
        compute(bo, CPB - 1, vrA)
        return carry

    lax.fori_loop(0, NBLK, block, 0)
    wait_sc(vwA, ssA, GA)
    wait_sc(vwB, ssB, GB)
    plsc.subcore_barrier()
    pltpu.sync_copy(acc_sh.at[pl.ds(rows0, RPS)],
                    accp_hbm.at[cid, pl.ds(rows0, RPS)])

    @pl.when(sid == NS - 1)
    def _wb_tail():
        t0 = pl.multiple_of(NS * RPS, 8)
        pltpu.sync_copy(acc_sh.at[pl.ds(t0, RTAIL)],
                        accp_hbm.at[cid, pl.ds(t0, RTAIL)])


def _agg(v, src3, dstA, dstB, scores, pmax, zeros):
    f = pl.kernel(
        _agg_body,
        out_type=jax.ShapeDtypeStruct((NC, N, AW), jnp.float32),
        mesh=_MESH,
        compiler_params=_SC_PARAMS,
        scratch_types=[
            pltpu.VMEM((CPB, G), jnp.int32),
            pltpu.VMEM((CPB, GA), jnp.int32),
            pltpu.VMEM((CPB, GB), jnp.int32),
            pltpu.VMEM((CPB * G,), jnp.float32),
            pltpu.VMEM((G,), jnp.float32),
            pltpu.VMEM((G, H), jnp.float32),
            pltpu.VMEM((G, H), jnp.float32),
            pltpu.VMEM((GA, AW), jnp.float32),
            pltpu.VMEM((GB, AW), jnp.float32),
            pltpu.VMEM((NW, L), jnp.float32),
            pltpu.VMEM_SHARED((N, AW), jnp.float32),
            pltpu.SemaphoreType.DMA,
            pltpu.SemaphoreType.DMA,
            pltpu.SemaphoreType.DMA,
            pltpu.SemaphoreType.DMA,
        ],
    )
    return f(v, src3, dstA, dstB, scores, pmax, zeros)


def _epi_body(accp_hbm, s_hbm, h_hbm, a0_v, a1_v, sv_v, hv_v, s0, s1, s2):
    wid = _wid()

    def step(k, carry):
        t = wid + k * NW

        @pl.when(t < NRC)
        def _():
            r0 = pl.multiple_of(t * RC, 8)
            cp0 = pltpu.async_copy(accp_hbm.at[0, pl.ds(r0, RC)], a0_v, s0)
            cp1 = pltpu.async_copy(accp_hbm.at[1, pl.ds(r0, RC)], a1_v, s1)
            cp2 = pltpu.async_copy(s_hbm.at[pl.ds(r0, RC)], sv_v, s2)
            cp0.wait()
            cp1.wait()
            cp2.wait()
            def rloop(r, c):
                d0 = a0_v[r, pl.ds(H, L)]
                d1 = a1_v[r, pl.ds(H, L)]
                rden = 1.0 / (d0 + d1 + 1e-16)
                for cc in range(8):
                    num = (a0_v[r, pl.ds(cc * L, L)]
                           + a1_v[r, pl.ds(cc * L, L)])
                    hv_v[r, pl.ds(cc * L, L)] = jnp.maximum(
                        sv_v[r, pl.ds(cc * L, L)] + num * rden, 0.0)
                return c

            lax.fori_loop(0, RC, rloop, 0)
            pltpu.sync_copy(hv_v, h_hbm.at[pl.ds(r0, RC)])

        return carry

    lax.fori_loop(0, EPI_K, step, 0)


def _epi(accp, s):
    f = pl.kernel(
        _epi_body,
        out_type=jax.ShapeDtypeStruct((N, H), jnp.float32),
        mesh=_MESH,
        compiler_params=_SC_PARAMS,
        scratch_types=[
            pltpu.VMEM((RC, AW), jnp.float32),
            pltpu.VMEM((RC, AW), jnp.float32),
            pltpu.VMEM((RC, H), jnp.float32),
            pltpu.VMEM((RC, H), jnp.float32),
            pltpu.SemaphoreType.DMA,
            pltpu.SemaphoreType.DMA,
            pltpu.SemaphoreType.DMA,
        ],
    )
    return f(accp, s)


def _sel_body(h_hbm, idx_hbm, out_hbm, idx_v, rows_v, sem):
    wid = _wid()
    base = pl.multiple_of(wid * SPW, 8)
    pltpu.sync_copy(idx_hbm.at[pl.ds(base, SPW)], idx_v)
    pltpu.async_copy(h_hbm.at[idx_v], rows_v, sem).wait()
    pltpu.sync_copy(rows_v, out_hbm.at[pl.ds(base, SPW)])


def _sel(h, idx):
    f = pl.kernel(
        _sel_body,
        out_type=jax.ShapeDtypeStruct((B, H), jnp.float32),
        mesh=_MESH,
        compiler_params=_SC_PARAMS,
        scratch_types=[
            pltpu.VMEM((SPW,), jnp.int32),
            pltpu.VMEM((SPW, H), jnp.float32),
            pltpu.SemaphoreType.DMA,
        ],
    )
    return f(h, idx)


def _layer(h, src3, dst3, dstA, dstB, wT, b2d, zeros):
    q, k, v, s = _proj(h, wT, b2d)
    scores, pmax = _score(q, k, src3, dst3)
    accp = _agg(v, src3, dstA, dstB, scores, pmax, zeros)
    return _epi(accp, s)


def kernel(x, edge_index, idx,
           Wq0, bq0, Wk0, bk0, Wv0, bv0, Ws0, bs0,
           Wq1, bq1, Wk1, bk1, Wv1, bv1, Ws1, bs1,
           W1, b1, W2, b2, W3, b3):
    src3 = edge_index[0].reshape(NW, NCHUNK, G)
    dst3 = edge_index[1].reshape(NW, NCHUNK, G)
    dstA = dst3[:, :, :GA]
    dstB = dst3[:, :, GA:]
    zeros = jnp.zeros((N, AW), jnp.float32)
    w0T = jnp.concatenate([Wq0, Wk0, Wv0, Ws0], axis=0).T
    b0 = jnp.concatenate([bq0, bk0, bv0, bs0]).reshape(1, 4 * H)
    w1T = jnp.concatenate([Wq1, Wk1, Wv1, Ws1], axis=0).T
    b1c = jnp.concatenate([bq1, bk1, bv1, bs1]).reshape(1, 4 * H)
    h = _layer(x, src3, dst3, dstA, dstB, w0T, b0, zeros)
    h = _layer(h, src3, dst3, dstA, dstB, w1T, b1c, zeros)
    hsel = _sel(h, idx)
    return _mlp(hsel, W1, b1, W2, b2, W3, b3)
